# baseline (speedup 1.0000x reference)
import jax
import jax.numpy as jnp
from jax import lax
from jax.experimental import pallas as pl
from jax.experimental.pallas import tpu as pltpu
from jax.experimental.pallas import tpu_sc as plsc

NC = 2
NS = 16
L = 16
NW = NC * NS
B = 16384
D = 32
GW = 128
BPW = B // NW
CH = 128
NCH = BPW // CH


def _mf_body(uid_h, iid_h, utab_h, itab_h, ubias_h, ibias_h, mean_h, out_h,
             uix, iix, ug, ig, uo, io, urows, irows, ub, ib, ov, mv,
             sem, bsem):
    c = lax.axis_index("c")
    s = lax.axis_index("s")
    wid = s * NC + c
    base = wid * BPW

    for j in range(NCH):
        pltpu.sync_copy(uid_h.at[pl.ds(base + j * CH, CH)], uix.at[j])
        pltpu.sync_copy(iid_h.at[pl.ds(base + j * CH, CH)], iix.at[j])
    pltpu.sync_copy(mean_h, mv)
    for j in range(NCH):
        for t in range(CH // L):
            sl = pl.ds(t * L, L)
            uv = uix[j, sl]
            iv = iix[j, sl]
            ug[j, sl] = uv >> 2
            ig[j, sl] = iv >> 2
            uo[j, sl] = (uv & 3) << 5
            io[j, sl] = (iv & 3) << 5

    bcopies = []
    for j in range(NCH):
        sl = pl.ds(j * CH, CH)
        bcopies.append(pltpu.async_copy(ubias_h.at[uix.at[j]], ub.at[sl], bsem))
        bcopies.append(pltpu.async_copy(ibias_h.at[iix.at[j]], ib.at[sl], bsem))
    for cp in bcopies:
        cp.wait()

    mean_vec = mv[...]

    for j in range(NCH):
        cpu = pltpu.async_copy(utab_h.at[ug.at[j]], urows, sem)
        cpi = pltpu.async_copy(itab_h.at[ig.at[j]], irows, sem)
        cpu.wait()
        cpi.wait()

        def group(gg, carry, j=j):
            s16 = pl.ds(j * CH + gg * L, L)
            riota = lax.iota(jnp.int32, L) + gg * L
            ou = uo[j, pl.ds(gg * L, L)]
            oi = io[j, pl.ds(gg * L, L)]
            acc = ub[s16] + ib[s16] + mean_vec
            for d in range(D):
                uvals = plsc.load_gather(urows, [riota, ou + d])
                ivals = plsc.load_gather(irows, [riota, oi + d])
                acc = acc + uvals * ivals
            ov[s16] = acc
            return carry

        lax.fori_loop(0, CH // L, group, 0)

    pltpu.sync_copy(ov, out_h.at[pl.ds(base, BPW)])


@jax.jit
def kernel(u_id, i_id, user_emb, item_emb, user_bias, item_bias, mean):
    mesh = plsc.VectorSubcoreMesh(core_axis_name="c", subcore_axis_name="s")
    f = pl.kernel(
        _mf_body,
        mesh=mesh,
        compiler_params=pltpu.CompilerParams(needs_layout_passes=False),
        out_type=jax.ShapeDtypeStruct((B,), jnp.float32),
        scratch_types=[
            pltpu.VMEM((NCH, CH), jnp.int32),
            pltpu.VMEM((NCH, CH), jnp.int32),
            pltpu.VMEM((NCH, CH), jnp.int32),
            pltpu.VMEM((NCH, CH), jnp.int32),
            pltpu.VMEM((NCH, CH), jnp.int32),
            pltpu.VMEM((NCH, CH), jnp.int32),
            pltpu.VMEM((CH, GW), jnp.float32),
            pltpu.VMEM((CH, GW), jnp.float32),
            pltpu.VMEM((BPW,), jnp.float32),
            pltpu.VMEM((BPW,), jnp.float32),
            pltpu.VMEM((BPW,), jnp.float32),
            pltpu.VMEM((L,), jnp.float32),
            pltpu.SemaphoreType.DMA,
            pltpu.SemaphoreType.DMA,
        ],
    )
    return f(u_id, i_id,
             user_emb.reshape(250000, GW),
             item_emb.reshape(250000, GW),
             user_bias.reshape(-1), item_bias.reshape(-1),
             jnp.broadcast_to(mean, (L,)))
